# baseline (speedup 1.0000x reference)
import jax
import jax.numpy as jnp
from jax import lax
from jax.experimental import pallas as pl
from jax.experimental.pallas import tpu as pltpu
from jax.experimental.pallas import tpu_sc as plsc

B, D, H, W, C = 2, 96, 160, 192, 2
N = B * D
ROWS = 16
NCHUNK = H // ROWS
NW = 32
SLICES_PER_W = N // NW
XBLKS = W // 16


def _warp_body(vol_hbm, flow_hbm, out_hbm, img0, img1, fy_v, fx_v, o0_v, o1_v,
               isem, fsems, osems):
    wid = lax.axis_index("s") * 2 + lax.axis_index("c")
    lane = lax.iota(jnp.int32, 16)
    lane_f = lane.astype(jnp.float32)

    def flow_copies(n, c, b):
        return (
            pltpu.make_async_copy(
                flow_hbm.at[n, 0, pl.ds(c * ROWS, ROWS)], fy_v.at[b], fsems[b]),
            pltpu.make_async_copy(
                flow_hbm.at[n, 1, pl.ds(c * ROWS, ROWS)], fx_v.at[b], fsems[b]),
        )

    def out_copies(n, c, b):
        return (
            pltpu.make_async_copy(
                o0_v.at[b], out_hbm.at[n, 0, pl.ds(c * ROWS, ROWS)], osems[b]),
            pltpu.make_async_copy(
                o1_v.at[b], out_hbm.at[n, 1, pl.ds(c * ROWS, ROWS)], osems[b]),
        )

    def compute_chunk(c, b):
        @plsc.parallel_loop(0, ROWS, unroll=2)
        def _row(r):
            y = c * ROWS + r
            yf = (jnp.zeros((16,), jnp.int32) + y).astype(jnp.float32)
            for j in range(XBLKS):
                fy = fy_v[b, r, pl.ds(j * 16, 16)]
                fx = fx_v[b, r, pl.ds(j * 16, 16)]
                qy = yf - fy
                qx = (lane_f - fx) + float(j * 16)
                qyc = jnp.minimum(jnp.maximum(qy, 0.0), float(H - 2))
                qxc = jnp.minimum(jnp.maximum(qx, 0.0), float(W - 2))
                y0 = qyc.astype(jnp.int32)
                x0 = qxc.astype(jnp.int32)
                ay = jnp.minimum(jnp.maximum(qy - y0.astype(jnp.float32), 0.0), 1.0)
                ax = jnp.minimum(jnp.maximum(qx - x0.astype(jnp.float32), 0.0), 1.0)
                x1 = x0 + 1
                y1 = y0 + 1
                tl0 = plsc.load_gather(img0, [y0, x0])
                tl1 = plsc.load_gather(img1, [y0, x0])
                tr0 = plsc.load_gather(img0, [y0, x1])
                tr1 = plsc.load_gather(img1, [y0, x1])
                bl0 = plsc.load_gather(img0, [y1, x0])
                bl1 = plsc.load_gather(img1, [y1, x0])
                br0 = plsc.load_gather(img0, [y1, x1])
                br1 = plsc.load_gather(img1, [y1, x1])
                top0 = tl0 + ax * (tr0 - tl0)
                top1 = tl1 + ax * (tr1 - tl1)
                bot0 = bl0 + ax * (br0 - bl0)
                bot1 = bl1 + ax * (br1 - bl1)
                o0_v[b, r, pl.ds(j * 16, 16)] = top0 + ay * (bot0 - top0)
                o1_v[b, r, pl.ds(j * 16, 16)] = top1 + ay * (bot1 - top1)

    @pl.loop(0, SLICES_PER_W)
    def _slice(k):
        n = wid * SLICES_PER_W + k
        ic0 = pltpu.make_async_copy(vol_hbm.at[n, :, 0], img0, isem)
        ic1 = pltpu.make_async_copy(vol_hbm.at[n, :, 1], img1, isem)
        ic0.start()
        ic1.start()
        for cp in flow_copies(n, 0, 0):
            cp.start()
        ic0.wait()
        ic1.wait()

        @pl.loop(0, NCHUNK, step=2)
        def _chunk(c0):
            for b in range(2):
                c = c0 + b
                if b == 0:
                    for cp in flow_copies(n, c + 1, 1):
                        cp.start()
                else:
                    @pl.when(c0 < NCHUNK - 2)
                    def _():
                        for cp in flow_copies(n, c + 1, 0):
                            cp.start()

                for cp in flow_copies(n, c, b):
                    cp.wait()

                @pl.when(c0 >= 2)
                def _():
                    for cp in out_copies(n, c - 2, b):
                        cp.wait()

                compute_chunk(c, b)
                for cp in out_copies(n, c, b):
                    cp.start()

        for b in range(2):
            for cp in out_copies(n, NCHUNK - 2 + b, b):
                cp.wait()


def kernel(vol, flow):
    vol_p = jnp.transpose(vol.reshape(N, H, W, C), (0, 1, 3, 2))
    flow_p = jnp.transpose(flow.reshape(N, H, W, 3), (0, 3, 1, 2))
    run = pl.kernel(
        _warp_body,
        out_type=jax.ShapeDtypeStruct((N, C, H, W), jnp.float32),
        mesh=plsc.VectorSubcoreMesh(core_axis_name="c", subcore_axis_name="s"),
        compiler_params=pltpu.CompilerParams(
            needs_layout_passes=False, use_tc_tiling_on_sc=False),
        scratch_types=[
            pltpu.VMEM((H, W), jnp.float32),
            pltpu.VMEM((H, W), jnp.float32),
            pltpu.VMEM((2, ROWS, W), jnp.float32),
            pltpu.VMEM((2, ROWS, W), jnp.float32),
            pltpu.VMEM((2, ROWS, W), jnp.float32),
            pltpu.VMEM((2, ROWS, W), jnp.float32),
            pltpu.SemaphoreType.DMA,
            [pltpu.SemaphoreType.DMA, pltpu.SemaphoreType.DMA],
            [pltpu.SemaphoreType.DMA, pltpu.SemaphoreType.DMA],
        ],
    )
    out = run(vol_p, flow_p)
    return jnp.transpose(out, (0, 2, 3, 1)).reshape(B, D, H, W, C)


# speedup vs baseline: 1.1573x; 1.1573x over previous
import jax
import jax.numpy as jnp
from jax import lax
from jax.experimental import pallas as pl
from jax.experimental.pallas import tpu as pltpu
from jax.experimental.pallas import tpu_sc as plsc

B, D, H, W, C = 2, 96, 160, 192, 2
N = B * D
ROWS = 16
NCHUNK = H // ROWS
NW = 32
SLICES_PER_W = N // NW
XBLKS = W // 16


def _warp_body(vol_hbm, flow_hbm, out_hbm, img0, img1, fy_v, fx_v, o0_v, o1_v,
               isem, fsems, osems):
    wid = lax.axis_index("s") * 2 + lax.axis_index("c")
    lane = lax.iota(jnp.int32, 16)
    lane_f = lane.astype(jnp.float32)

    def flow_copies(n, c, b):
        return (
            pltpu.make_async_copy(
                flow_hbm.at[n, 0, pl.ds(c * ROWS, ROWS)], fy_v.at[b], fsems[b]),
            pltpu.make_async_copy(
                flow_hbm.at[n, 1, pl.ds(c * ROWS, ROWS)], fx_v.at[b], fsems[b]),
        )

    def out_copies(n, c, b):
        return (
            pltpu.make_async_copy(
                o0_v.at[b], out_hbm.at[n, 0, pl.ds(c * ROWS, ROWS)], osems[b]),
            pltpu.make_async_copy(
                o1_v.at[b], out_hbm.at[n, 1, pl.ds(c * ROWS, ROWS)], osems[b]),
        )

    def compute_chunk(c, b):
        @plsc.parallel_loop(0, ROWS, unroll=2)
        def _row(r):
            y = c * ROWS + r
            yf = (jnp.zeros((16,), jnp.int32) + y).astype(jnp.float32)
            for j in range(XBLKS):
                fy = fy_v[b, r, pl.ds(j * 16, 16)]
                fx = fx_v[b, r, pl.ds(j * 16, 16)]
                qy = yf - fy
                qx = (lane_f - fx) + float(j * 16)
                qyc = jnp.minimum(jnp.maximum(qy, 0.0), float(H - 2))
                qxc = jnp.minimum(jnp.maximum(qx, 0.0), float(W - 2))
                y0 = qyc.astype(jnp.int32)
                x0 = qxc.astype(jnp.int32)
                ay = jnp.minimum(jnp.maximum(qy - y0.astype(jnp.float32), 0.0), 1.0)
                ax = jnp.minimum(jnp.maximum(qx - x0.astype(jnp.float32), 0.0), 1.0)
                x1 = x0 + 1
                y1 = y0 + 1
                tl0 = plsc.load_gather(img0, [y0, x0])
                tl1 = plsc.load_gather(img1, [y0, x0])
                tr0 = plsc.load_gather(img0, [y0, x1])
                tr1 = plsc.load_gather(img1, [y0, x1])
                bl0 = plsc.load_gather(img0, [y1, x0])
                bl1 = plsc.load_gather(img1, [y1, x0])
                br0 = plsc.load_gather(img0, [y1, x1])
                br1 = plsc.load_gather(img1, [y1, x1])
                top0 = tl0 + ax * (tr0 - tl0)
                top1 = tl1 + ax * (tr1 - tl1)
                bot0 = bl0 + ax * (br0 - bl0)
                bot1 = bl1 + ax * (br1 - bl1)
                o0_v[b, r, pl.ds(j * 16, 16)] = top0 + ay * (bot0 - top0)
                o1_v[b, r, pl.ds(j * 16, 16)] = top1 + ay * (bot1 - top1)

    @pl.loop(0, SLICES_PER_W)
    def _slice(k):
        n = wid * SLICES_PER_W + k
        ic0 = pltpu.make_async_copy(vol_hbm.at[n, 0], img0, isem)
        ic1 = pltpu.make_async_copy(vol_hbm.at[n, 1], img1, isem)
        ic0.start()
        ic1.start()
        for cp in flow_copies(n, 0, 0):
            cp.start()
        ic0.wait()
        ic1.wait()

        @pl.loop(0, NCHUNK, step=2)
        def _chunk(c0):
            for b in range(2):
                c = c0 + b
                if b == 0:
                    for cp in flow_copies(n, c + 1, 1):
                        cp.start()
                else:
                    @pl.when(c0 < NCHUNK - 2)
                    def _():
                        for cp in flow_copies(n, c + 1, 0):
                            cp.start()

                for cp in flow_copies(n, c, b):
                    cp.wait()

                @pl.when(c0 >= 2)
                def _():
                    for cp in out_copies(n, c - 2, b):
                        cp.wait()

                compute_chunk(c, b)
                for cp in out_copies(n, c, b):
                    cp.start()

        for b in range(2):
            for cp in out_copies(n, NCHUNK - 2 + b, b):
                cp.wait()


def kernel(vol, flow):
    vol_p = jnp.transpose(vol.reshape(N, H, W, C), (0, 3, 1, 2))
    flow_p = jnp.transpose(flow.reshape(N, H, W, 3), (0, 3, 1, 2))
    run = pl.kernel(
        _warp_body,
        out_type=jax.ShapeDtypeStruct((N, C, H, W), jnp.float32),
        mesh=plsc.VectorSubcoreMesh(core_axis_name="c", subcore_axis_name="s"),
        compiler_params=pltpu.CompilerParams(
            needs_layout_passes=False, use_tc_tiling_on_sc=False),
        scratch_types=[
            pltpu.VMEM((H, W), jnp.float32),
            pltpu.VMEM((H, W), jnp.float32),
            pltpu.VMEM((2, ROWS, W), jnp.float32),
            pltpu.VMEM((2, ROWS, W), jnp.float32),
            pltpu.VMEM((2, ROWS, W), jnp.float32),
            pltpu.VMEM((2, ROWS, W), jnp.float32),
            pltpu.SemaphoreType.DMA,
            [pltpu.SemaphoreType.DMA, pltpu.SemaphoreType.DMA],
            [pltpu.SemaphoreType.DMA, pltpu.SemaphoreType.DMA],
        ],
    )
    out = run(vol_p, flow_p)
    return jnp.transpose(out, (0, 2, 3, 1)).reshape(B, D, H, W, C)
